# baseline (speedup 1.0000x reference)
import functools

import jax
import jax.numpy as jnp
from jax import lax
from jax.experimental import pallas as pl
from jax.experimental.pallas import tpu as pltpu
from jax.experimental.pallas import tpu_sc as plsc

EMBED = 256
BATCH_BLOCK = 32
LANES = 16
NUM_WORKERS = 32


def _row_fill_body(row_ref):
    row_ref[...] = jnp.zeros(row_ref.shape, jnp.float32)


def _tc_row_emb(b, r, dtype):
    return pl.pallas_call(
        _row_fill_body,
        grid=(b // BATCH_BLOCK,),
        out_specs=pl.BlockSpec((BATCH_BLOCK, r, EMBED), lambda i: (i, 0, 0)),
        out_shape=jax.ShapeDtypeStruct((b, r, EMBED), dtype),
    )()


def _make_sc_col_fill(b, c):
    bpw = b // NUM_WORKERS
    mesh = plsc.VectorSubcoreMesh(core_axis_name="c", subcore_axis_name="s")

    @functools.partial(
        pl.kernel,
        out_type=jax.ShapeDtypeStruct((b, c, EMBED), jnp.float32),
        mesh=mesh,
        scratch_types=[
            pltpu.VMEM((c, EMBED), jnp.float32),
            pltpu.SemaphoreType.DMA,
        ],
        compiler_params=pltpu.CompilerParams(
            use_tc_tiling_on_sc=False, needs_layout_passes=False
        ),
    )
    def _sc_col_fill(out_hbm, pattern_v, sem):
        zeros16 = jnp.zeros((LANES,), jnp.float32)

        def zrow(j, carry):
            for k in range(EMBED // LANES):
                pattern_v[j, pl.ds(LANES * k, LANES)] = zeros16
            return carry

        lax.fori_loop(0, c, zrow, 0)

        ones16 = jnp.ones((LANES,), jnp.float32)
        for t in range(pl.cdiv(c, LANES)):
            idx = lax.iota(jnp.int32, LANES) + (LANES * t)
            plsc.store_scatter(pattern_v, [idx, idx], ones16, mask=idx < c)

        wid = lax.axis_index("s") * 2 + lax.axis_index("c")
        base = wid * bpw

        def fire(i, carry):
            pltpu.make_async_copy(pattern_v, out_hbm.at[base + i], sem).start()
            return carry

        def drain(i, carry):
            pltpu.make_async_copy(pattern_v, out_hbm.at[base], sem).wait()
            return carry

        lax.fori_loop(0, bpw, fire, 0)
        lax.fori_loop(0, bpw, drain, 0)

    return _sc_col_fill


def kernel(cost_matrix):
    b, r, c = cost_matrix.shape
    row_emb = _tc_row_emb(b, r, cost_matrix.dtype)
    col_emb = _make_sc_col_fill(b, c)()
    return (row_emb, col_emb, cost_matrix)


# speedup vs baseline: 1.7771x; 1.7771x over previous
import functools

import jax
import jax.numpy as jnp
from jax import lax
from jax.experimental import pallas as pl
from jax.experimental.pallas import tpu as pltpu
from jax.experimental.pallas import tpu_sc as plsc

EMBED = 256
BATCH_BLOCK = 32
NUM_WORKERS = 32


def _eye_body(eye_ref):
    n, e = eye_ref.shape
    i = lax.broadcasted_iota(jnp.int32, (n, e), 0)
    j = lax.broadcasted_iota(jnp.int32, (n, e), 1)
    eye_ref[...] = (i == j).astype(jnp.float32)


def _row_fill_body(row_ref):
    row_ref[...] = jnp.zeros(row_ref.shape, jnp.float32)


def _make_sc_col_fill(b, c):
    bpw = b // NUM_WORKERS
    mesh = plsc.VectorSubcoreMesh(core_axis_name="c", subcore_axis_name="s")

    @functools.partial(
        pl.kernel,
        out_type=jax.ShapeDtypeStruct((b, c, EMBED), jnp.float32),
        mesh=mesh,
        scratch_types=[
            pltpu.VMEM((c, EMBED), jnp.float32),
            pltpu.SemaphoreType.DMA,
        ],
    )
    def _sc_col_fill(eye_hbm, out_hbm, pattern_v, sem):
        pltpu.sync_copy(eye_hbm, pattern_v)
        wid = lax.axis_index("s") * 2 + lax.axis_index("c")
        base = wid * bpw

        def fire(i, carry):
            pltpu.make_async_copy(pattern_v, out_hbm.at[base + i], sem).start()
            return carry

        def drain(i, carry):
            pltpu.make_async_copy(pattern_v, out_hbm.at[base], sem).wait()
            return carry

        lax.fori_loop(0, bpw, fire, 0)
        lax.fori_loop(0, bpw, drain, 0)

    return _sc_col_fill


def kernel(cost_matrix):
    b, r, c = cost_matrix.shape
    eye = pl.pallas_call(
        _eye_body,
        out_shape=jax.ShapeDtypeStruct((c, EMBED), cost_matrix.dtype),
    )()
    row_emb = pl.pallas_call(
        _row_fill_body,
        grid=(b // BATCH_BLOCK,),
        out_specs=pl.BlockSpec((BATCH_BLOCK, r, EMBED), lambda i: (i, 0, 0)),
        out_shape=jax.ShapeDtypeStruct((b, r, EMBED), cost_matrix.dtype),
    )()
    col_emb = _make_sc_col_fill(b, c)(eye)
    return (row_emb, col_emb, cost_matrix)
